# Initial kernel scaffold; baseline (speedup 1.0000x reference)
import jax
import jax.numpy as jnp
from jax.experimental import pallas as pl


def kernel(x, edge_index, W1, b1, W2, b2):
    raise NotImplementedError("write your pallas kernel here")


# speedup vs baseline: 30.7730x; 30.7730x over previous
import dataclasses
import functools

import jax
import jax.numpy as jnp
from jax import lax
from jax.experimental import pallas as pl
from jax.experimental.pallas import tpu as pltpu
from jax.experimental.pallas import tpu_sc as plsc

NC = 2
NS = 16
L = 16
K = 125

_MESH = plsc.VectorSubcoreMesh(
    core_axis_name="c", subcore_axis_name="s", num_cores=NC, num_subcores=NS
)

_NO_LAYOUT = pltpu.CompilerParams()
if "needs_layout_passes" in pltpu.CompilerParams.__dataclass_fields__:
    _NO_LAYOUT = dataclasses.replace(_NO_LAYOUT, needs_layout_passes=False)

_LINEAR = dataclasses.replace(
    pltpu.CompilerParams(), use_tc_tiling_on_sc=False)


def _f32(*shape):
    return jax.ShapeDtypeStruct(shape, jnp.float32)


def _sc_degree(dst_flat):
    nw, epw = dst_flat.shape
    n = 10000

    @functools.partial(
        pl.kernel,
        out_type=_f32(nw, n),
        mesh=_MESH,
        scratch_types=[
            pltpu.VMEM((epw,), jnp.int32),
            pltpu.VMEM((n,), jnp.float32),
        ],
        compiler_params=_NO_LAYOUT,
    )
    def deg_kernel(dst_hbm, hist_hbm, dstv, histv):
        c = lax.axis_index("c")
        s = lax.axis_index("s")
        wid = c * NS + s
        pltpu.sync_copy(dst_hbm.at[wid], dstv)

        @pl.loop(0, n // L)
        def _zero(i):
            histv[pl.ds(i * L, L)] = jnp.zeros((L,), jnp.float32)

        ones = jnp.ones((L,), jnp.float32)

        @pl.loop(0, epw // L)
        def _acc(t):
            idx = dstv[pl.ds(t * L, L)]
            plsc.addupdate_scatter(histv, [idx], ones)

        pltpu.sync_copy(histv, hist_hbm.at[wid])

    return deg_kernel(dst_flat)


def _sc_spmm1(ya, yb, src2, dst2):
    n, f = ya.shape
    _, ch, _ = src2.shape
    rpt = n // NS

    @functools.partial(
        pl.kernel,
        out_type=(_f32(n, f), _f32(n, f)),
        mesh=_MESH,
        scratch_types=[
            pltpu.VMEM((ch, K), jnp.int32),
            pltpu.VMEM((ch, K), jnp.int32),
            pltpu.VMEM((K, f), jnp.float32),
            pltpu.VMEM((K, f), jnp.float32),
            pltpu.VMEM_SHARED((n, f), jnp.float32),
            pltpu.SemaphoreType.DMA,
        ],
        compiler_params=_LINEAR,
    )
    def spmm1_kernel(ya_hbm, yb_hbm, src_hbm, dst_hbm, oa_hbm, ob_hbm,
                     srcv, dstv, rows0, rows1, acc, sem):
        c = lax.axis_index("c")
        s = lax.axis_index("s")
        pltpu.sync_copy(src_hbm.at[s], srcv)
        pltpu.sync_copy(dst_hbm.at[s], dstv)

        def run(tbl, out):
            pltpu.sync_copy(tbl.at[pl.ds(s * rpt, rpt)],
                            acc.at[pl.ds(s * rpt, rpt)])
            plsc.subcore_barrier()

            @pl.loop(0, ch, step=2)
            def _chunk(k):
                g0 = pltpu.async_copy(tbl.at[srcv.at[k]], rows0, sem)
                g1 = pltpu.async_copy(tbl.at[srcv.at[k + 1]], rows1, sem)
                g0.wait()
                pltpu.sync_copy(rows0, acc.at[dstv.at[k]], add=True)
                g1.wait()
                pltpu.sync_copy(rows1, acc.at[dstv.at[k + 1]], add=True)

            plsc.subcore_barrier()
            pltpu.sync_copy(acc.at[pl.ds(s * rpt, rpt)],
                            out.at[pl.ds(s * rpt, rpt)])

        @pl.when(c == 0)
        def _():
            run(ya_hbm, oa_hbm)

        @pl.when(c == 1)
        def _():
            run(yb_hbm, ob_hbm)

    return spmm1_kernel(ya, yb, src2, dst2)


def _sc_spmm2(y2, src3, dst3):
    n, f = y2.shape
    nw, ch, _ = src3.shape
    rpt = n // NS

    @functools.partial(
        pl.kernel,
        out_type=_f32(NC, n, f),
        mesh=_MESH,
        scratch_types=[
            pltpu.VMEM((ch, K), jnp.int32),
            pltpu.VMEM((ch, K), jnp.int32),
            pltpu.VMEM((K, f), jnp.float32),
            pltpu.VMEM((K, f), jnp.float32),
            pltpu.VMEM_SHARED((n, f), jnp.float32),
            pltpu.SemaphoreType.DMA,
        ],
        compiler_params=_LINEAR,
    )
    def spmm2_kernel(y_hbm, src_hbm, dst_hbm, o_hbm,
                     srcv, dstv, rows0, rows1, acc, sem):
        c = lax.axis_index("c")
        s = lax.axis_index("s")
        wid = c * NS + s
        pltpu.sync_copy(src_hbm.at[wid], srcv)
        pltpu.sync_copy(dst_hbm.at[wid], dstv)
        pltpu.sync_copy(y_hbm.at[pl.ds(s * rpt, rpt)],
                        acc.at[pl.ds(s * rpt, rpt)])
        plsc.subcore_barrier()

        @pl.loop(0, ch, step=2)
        def _chunk(k):
            g0 = pltpu.async_copy(y_hbm.at[srcv.at[k]], rows0, sem)
            g1 = pltpu.async_copy(y_hbm.at[srcv.at[k + 1]], rows1, sem)
            g0.wait()
            pltpu.sync_copy(rows0, acc.at[dstv.at[k]], add=True)
            g1.wait()
            pltpu.sync_copy(rows1, acc.at[dstv.at[k + 1]], add=True)

        plsc.subcore_barrier()
        pltpu.sync_copy(acc.at[pl.ds(s * rpt, rpt)],
                        o_hbm.at[c, pl.ds(s * rpt, rpt)])

    return spmm2_kernel(y2, src3, dst3)


def _tc_matmul(x, w):
    n, d = x.shape
    h = w.shape[1]
    bn = 2000

    def body(x_ref, w_ref, o_ref):
        o_ref[...] = lax.dot_general(
            x_ref[...], w_ref[...], (((1,), (0,)), ((), ())),
            precision=lax.Precision.HIGHEST,
            preferred_element_type=jnp.float32)

    return pl.pallas_call(
        body,
        grid=(n // bn,),
        in_specs=[
            pl.BlockSpec((bn, d), lambda i: (i, 0)),
            pl.BlockSpec((d, h), lambda i: (0, 0)),
        ],
        out_specs=pl.BlockSpec((bn, h), lambda i: (i, 0)),
        out_shape=_f32(n, h),
    )(x, w)


def _tc_scale_split(xw, hist):
    n, h = xw.shape
    f = h // 2

    def body(xw_ref, hist_ref, ya_ref, yb_ref, dinv_ref):
        deg = jnp.sum(hist_ref[...], axis=0) + 1.0
        dinv = lax.rsqrt(deg)[:, None]
        y = xw_ref[...] * dinv
        ya_ref[...] = y[:, :f]
        yb_ref[...] = y[:, f:]
        dinv_ref[...] = dinv

    return pl.pallas_call(
        body,
        out_shape=(_f32(n, f), _f32(n, f), _f32(n, 1)),
    )(xw, hist)


def _tc_mid(oa, ob, dinv, b1, w2p):
    n, f = oa.shape
    fp = w2p.shape[1]

    def body(oa_ref, ob_ref, dinv_ref, b1_ref, w2_ref, y2_ref):
        agg = jnp.concatenate([oa_ref[...], ob_ref[...]], axis=1)
        dinv = dinv_ref[...]
        h = jnp.maximum(agg * dinv + b1_ref[...], 0.0)
        y2 = lax.dot_general(
            h, w2_ref[...], (((1,), (0,)), ((), ())),
            precision=lax.Precision.HIGHEST,
            preferred_element_type=jnp.float32)
        y2_ref[...] = y2 * dinv

    return pl.pallas_call(body, out_shape=_f32(n, fp))(oa, ob, dinv, b1, w2p)


def _tc_final(o2, y2, dinv, b2p, c_out):
    n, fp = y2.shape

    def body(o2_ref, y2_ref, dinv_ref, b2_ref, out_ref):
        agg = o2_ref[0] + o2_ref[1] - y2_ref[...]
        res = agg * dinv_ref[...] + b2_ref[...]
        out_ref[...] = res[:, :c_out]

    return pl.pallas_call(body, out_shape=_f32(n, c_out))(o2, y2, dinv, b2p)


def kernel(x, edge_index, W1, b1, W2, b2):
    n, d = x.shape
    h = W1.shape[1]
    c_out = W2.shape[1]
    e = edge_index.shape[1]
    fp = 16

    src = edge_index[0]
    dst = edge_index[1]
    src2 = src.reshape(NS, e // NS // K, K)
    dst2 = dst.reshape(NS, e // NS // K, K)
    src3 = src.reshape(NC * NS, e // (NC * NS) // K, K)
    dst3 = dst.reshape(NC * NS, e // (NC * NS) // K, K)
    dstf = dst.reshape(NC * NS, e // (NC * NS))

    w2p = jnp.pad(W2, ((0, 0), (0, fp - c_out)))
    b1r = b1.reshape(1, h)
    b2p = jnp.pad(b2, (0, fp - c_out)).reshape(1, fp)

    hist = _sc_degree(dstf)
    xw = _tc_matmul(x, W1)
    ya, yb, dinv = _tc_scale_split(xw, hist)
    oa, ob = _sc_spmm1(ya, yb, src2, dst2)
    y2 = _tc_mid(oa, ob, dinv, b1r, w2p)
    o2 = _sc_spmm2(y2, src3, dst3)
    out = _tc_final(o2, y2, dinv, b2p, c_out)
    return out, edge_index


# speedup vs baseline: 41.9438x; 1.3630x over previous
import dataclasses
import functools

import jax
import jax.numpy as jnp
from jax import lax
from jax.experimental import pallas as pl
from jax.experimental.pallas import tpu as pltpu
from jax.experimental.pallas import tpu_sc as plsc

NC = 2
NS = 16
L = 16
K = 125

_MESH = plsc.VectorSubcoreMesh(
    core_axis_name="c", subcore_axis_name="s", num_cores=NC, num_subcores=NS
)

_NO_LAYOUT = pltpu.CompilerParams()
if "needs_layout_passes" in pltpu.CompilerParams.__dataclass_fields__:
    _NO_LAYOUT = dataclasses.replace(_NO_LAYOUT, needs_layout_passes=False)

_LINEAR = dataclasses.replace(
    pltpu.CompilerParams(), use_tc_tiling_on_sc=False)


def _f32(*shape):
    return jax.ShapeDtypeStruct(shape, jnp.float32)


def _sc_degree(dst_flat):
    nw, epw = dst_flat.shape
    n = 10000

    @functools.partial(
        pl.kernel,
        out_type=_f32(nw, n),
        mesh=_MESH,
        scratch_types=[
            pltpu.VMEM((epw,), jnp.int32),
            pltpu.VMEM((n,), jnp.float32),
        ],
        compiler_params=_NO_LAYOUT,
    )
    def deg_kernel(dst_hbm, hist_hbm, dstv, histv):
        c = lax.axis_index("c")
        s = lax.axis_index("s")
        wid = c * NS + s
        pltpu.sync_copy(dst_hbm.at[wid], dstv)

        @pl.loop(0, n // L)
        def _zero(i):
            histv[pl.ds(i * L, L)] = jnp.zeros((L,), jnp.float32)

        ones = jnp.ones((L,), jnp.float32)

        @pl.loop(0, epw // L)
        def _acc(t):
            idx = dstv[pl.ds(t * L, L)]
            plsc.addupdate_scatter(histv, [idx], ones)

        pltpu.sync_copy(histv, hist_hbm.at[wid])

    return deg_kernel(dst_flat)


NBUF = 4


def _ring_spmm(tbl, out, srcv, dstv, rows, sems, acc, ch, s, rpt):
    pltpu.sync_copy(tbl.at[pl.ds(s * rpt, rpt)],
                    acc.at[pl.ds(s * rpt, rpt)])
    plsc.subcore_barrier()

    def fire(k, b):
        pltpu.async_copy(tbl.at[srcv.at[k]], rows[b], sems.at[b])

    def drain(k, b):
        pltpu.make_async_copy(tbl.at[srcv.at[k]], rows[b], sems.at[b]).wait()

    for b in range(NBUF):
        fire(b, b)

    @pl.loop(0, ch // NBUF - 1)
    def _group(m):
        for b in range(NBUF):
            k = m * NBUF + b
            drain(k, b)
            pltpu.sync_copy(rows[b], acc.at[dstv.at[k]], add=True)
            fire(k + NBUF, b)

    for b in range(NBUF):
        k = ch - NBUF + b
        drain(k, b)
        pltpu.sync_copy(rows[b], acc.at[dstv.at[k]], add=True)

    plsc.subcore_barrier()
    pltpu.sync_copy(acc.at[pl.ds(s * rpt, rpt)],
                    out.at[pl.ds(s * rpt, rpt)])


def _sc_spmm1(ya, yb, src2, dst2):
    n, f = ya.shape
    _, ch, _ = src2.shape
    rpt = n // NS

    @functools.partial(
        pl.kernel,
        out_type=(_f32(n, f), _f32(n, f)),
        mesh=_MESH,
        scratch_types=[
            pltpu.VMEM((ch, K), jnp.int32),
            pltpu.VMEM((ch, K), jnp.int32),
            [pltpu.VMEM((K, f), jnp.float32) for _ in range(NBUF)],
            pltpu.SemaphoreType.DMA((NBUF,)),
            pltpu.VMEM_SHARED((n, f), jnp.float32),
        ],
        compiler_params=_LINEAR,
    )
    def spmm1_kernel(ya_hbm, yb_hbm, src_hbm, dst_hbm, oa_hbm, ob_hbm,
                     srcv, dstv, rows, sems, acc):
        c = lax.axis_index("c")
        s = lax.axis_index("s")
        pltpu.sync_copy(src_hbm.at[s], srcv)
        pltpu.sync_copy(dst_hbm.at[s], dstv)

        @pl.when(c == 0)
        def _():
            _ring_spmm(ya_hbm, oa_hbm, srcv, dstv, rows, sems, acc,
                       ch, s, rpt)

        @pl.when(c == 1)
        def _():
            _ring_spmm(yb_hbm, ob_hbm, srcv, dstv, rows, sems, acc,
                       ch, s, rpt)

    return spmm1_kernel(ya, yb, src2, dst2)


def _sc_spmm2(y2, src3, dst3):
    n, f = y2.shape
    nw, ch, _ = src3.shape
    rpt = n // NS

    @functools.partial(
        pl.kernel,
        out_type=_f32(NC, n, f),
        mesh=_MESH,
        scratch_types=[
            pltpu.VMEM((ch, K), jnp.int32),
            pltpu.VMEM((ch, K), jnp.int32),
            [pltpu.VMEM((K, f), jnp.float32) for _ in range(NBUF)],
            pltpu.SemaphoreType.DMA((NBUF,)),
            pltpu.VMEM_SHARED((n, f), jnp.float32),
        ],
        compiler_params=_LINEAR,
    )
    def spmm2_kernel(y_hbm, src_hbm, dst_hbm, o_hbm,
                     srcv, dstv, rows, sems, acc):
        c = lax.axis_index("c")
        s = lax.axis_index("s")
        wid = c * NS + s
        pltpu.sync_copy(src_hbm.at[wid], srcv)
        pltpu.sync_copy(dst_hbm.at[wid], dstv)
        _ring_spmm(y_hbm, o_hbm.at[c], srcv, dstv, rows, sems, acc,
                   ch, s, rpt)

    return spmm2_kernel(y2, src3, dst3)


def _tc_matmul(x, w):
    n, d = x.shape
    h = w.shape[1]
    bn = 2000

    def body(x_ref, w_ref, o_ref):
        o_ref[...] = lax.dot_general(
            x_ref[...], w_ref[...], (((1,), (0,)), ((), ())),
            precision=lax.Precision.HIGHEST,
            preferred_element_type=jnp.float32)

    return pl.pallas_call(
        body,
        grid=(n // bn,),
        in_specs=[
            pl.BlockSpec((bn, d), lambda i: (i, 0)),
            pl.BlockSpec((d, h), lambda i: (0, 0)),
        ],
        out_specs=pl.BlockSpec((bn, h), lambda i: (i, 0)),
        out_shape=_f32(n, h),
    )(x, w)


def _tc_scale_split(xw, hist):
    n, h = xw.shape
    f = h // 2

    def body(xw_ref, hist_ref, ya_ref, yb_ref, dinv_ref):
        deg = jnp.sum(hist_ref[...], axis=0) + 1.0
        dinv = lax.rsqrt(deg)[:, None]
        y = xw_ref[...] * dinv
        ya_ref[...] = y[:, :f]
        yb_ref[...] = y[:, f:]
        dinv_ref[...] = dinv

    return pl.pallas_call(
        body,
        out_shape=(_f32(n, f), _f32(n, f), _f32(n, 1)),
    )(xw, hist)


def _tc_mid(oa, ob, dinv, b1, w2p):
    n, f = oa.shape
    fp = w2p.shape[1]

    def body(oa_ref, ob_ref, dinv_ref, b1_ref, w2_ref, y2_ref):
        agg = jnp.concatenate([oa_ref[...], ob_ref[...]], axis=1)
        dinv = dinv_ref[...]
        h = jnp.maximum(agg * dinv + b1_ref[...], 0.0)
        y2 = lax.dot_general(
            h, w2_ref[...], (((1,), (0,)), ((), ())),
            precision=lax.Precision.HIGHEST,
            preferred_element_type=jnp.float32)
        y2_ref[...] = y2 * dinv

    return pl.pallas_call(body, out_shape=_f32(n, fp))(oa, ob, dinv, b1, w2p)


def _tc_final(o2, y2, dinv, b2p, c_out):
    n, fp = y2.shape

    def body(o2_ref, y2_ref, dinv_ref, b2_ref, out_ref):
        agg = o2_ref[0] + o2_ref[1] - y2_ref[...]
        res = agg * dinv_ref[...] + b2_ref[...]
        out_ref[...] = res[:, :c_out]

    return pl.pallas_call(body, out_shape=_f32(n, c_out))(o2, y2, dinv, b2p)


def kernel(x, edge_index, W1, b1, W2, b2):
    n, d = x.shape
    h = W1.shape[1]
    c_out = W2.shape[1]
    e = edge_index.shape[1]
    fp = 16

    src = edge_index[0]
    dst = edge_index[1]
    src2 = src.reshape(NS, e // NS // K, K)
    dst2 = dst.reshape(NS, e // NS // K, K)
    src3 = src.reshape(NC * NS, e // (NC * NS) // K, K)
    dst3 = dst.reshape(NC * NS, e // (NC * NS) // K, K)
    dstf = dst.reshape(NC * NS, e // (NC * NS))

    w2p = jnp.pad(W2, ((0, 0), (0, fp - c_out)))
    b1r = b1.reshape(1, h)
    b2p = jnp.pad(b2, (0, fp - c_out)).reshape(1, fp)

    hist = _sc_degree(dstf)
    xw = _tc_matmul(x, W1)
    ya, yb, dinv = _tc_scale_split(xw, hist)
    oa, ob = _sc_spmm1(ya, yb, src2, dst2)
    y2 = _tc_mid(oa, ob, dinv, b1r, w2p)
    o2 = _sc_spmm2(y2, src3, dst3)
    out = _tc_final(o2, y2, dinv, b2p, c_out)
    return out, edge_index
